# baseline (speedup 1.0000x reference)
import functools

import jax
import jax.numpy as jnp
from jax import lax
from jax.experimental import pallas as pl
from jax.experimental.pallas import tpu as pltpu
from jax.experimental.pallas import tpu_sc as plsc

B = 128
V = 100000
VP = 102400
K = 2048
M = 4096
NB = 4096
CH = 10240
NCH = VP // CH
GRP = CH // 16
CHR = CH // 128
HKEY = 0x7FFFFFFF


def _sc_select(logits):
    info = plsc.get_sparse_core_info()
    nc, ns = info.num_cores, info.num_subcores
    nw = nc * ns
    rows_per_w = B // nw
    mesh = plsc.VectorSubcoreMesh(core_axis_name="c", subcore_axis_name="s")

    @functools.partial(
        pl.kernel,
        mesh=mesh,
        compiler_params=pltpu.CompilerParams(needs_layout_passes=False),
        out_type=[
            jax.ShapeDtypeStruct((B * M,), jnp.float32),
            jax.ShapeDtypeStruct((B * M,), jnp.int32),
            jax.ShapeDtypeStruct((B * 16,), jnp.float32),
        ],
        scratch_types=[
            pltpu.VMEM((VP // 128, 128), jnp.float32),
            pltpu.VMEM((NB,), jnp.int32),
            pltpu.VMEM((M,), jnp.float32),
            pltpu.VMEM((M,), jnp.int32),
            pltpu.VMEM((16,), jnp.float32),
            pltpu.SemaphoreType.DMA((4,)),
        ],
    )
    def k(logits_hbm, cv_hbm, ci_hbm, st_hbm, buf, hist, cv, ci, st, sem):
        wid = lax.axis_index("s") * nc + lax.axis_index("c")
        lanes = lax.iota(jnp.int32, 16)
        ones = jnp.ones((16,), jnp.int32)

        def zero_hist(i, c):
            hist[pl.ds(i * 16, 16)] = jnp.zeros((16,), jnp.int32)
            return c

        lax.fori_loop(0, NB // 16, zero_hist, 0)

        def bucket_of(x):
            bits = lax.bitcast_convert_type(x, jnp.int32)
            key = bits ^ ((bits >> 31) & HKEY)
            return (key >> 20) + 2048, key

        def start_copy(row2d, c):
            pltpu.make_async_copy(
                logits_hbm.at[pl.ds(row2d + c * CHR, CHR), :],
                buf.at[pl.ds(c * CHR, CHR), :], sem.at[c % 4]).start()

        def wait_copy(c):
            pltpu.make_async_copy(
                logits_hbm.at[pl.ds(0, CHR), :],
                buf.at[pl.ds(c * CHR, CHR), :], sem.at[c % 4]).wait()

        def row_body(r, _unused):
            row = wid * rows_per_w + r
            rowbase = row * (VP // 128)

            for cc in range(4):
                start_copy(rowbase, cc)

            def chunk1(c, vmax):
                @pl.when(c + 4 < NCH)
                def _():
                    start_copy(rowbase, c + 4)

                wait_copy(c)

                def body1(g, vmax):
                    gg = c * GRP + g
                    x = buf[gg >> 3, pl.ds((gg & 7) * 16, 16)]
                    bkt, _ = bucket_of(x)
                    plsc.addupdate_scatter(hist, [bkt], ones)
                    return jnp.maximum(vmax, x)

                return lax.fori_loop(0, GRP, body1, vmax, unroll=8)

            vmax = lax.fori_loop(
                0, NCH, chunk1, jnp.full((16,), -jnp.inf, jnp.float32))
            mx = jnp.max(vmax)

            def body2(t_rev, carry):
                acc, bnd, found = carry
                t = (NB // 16 - 1) - t_rev
                s = hist[pl.ds(t * 16, 16)]
                hist[pl.ds(t * 16, 16)] = jnp.zeros((16,), jnp.int32)
                cs = plsc.cumsum(s)
                total = jnp.max(cs)
                ssum = total - cs + s
                m = (acc + ssum) >= K
                cnt = jnp.max(plsc.all_reduce_population_count(m))
                newly = jnp.logical_and(jnp.logical_not(found), cnt > 0)
                bnd = jnp.where(newly, t * 16 + cnt - 1, bnd)
                found = jnp.logical_or(found, cnt > 0)
                return acc + total, bnd, found

            _, bnd, _ = lax.fori_loop(
                0, NB // 16, body2, (jnp.int32(0), jnp.int32(0), False))

            def body3(i, c):
                cv[pl.ds(i * 16, 16)] = jnp.full((16,), -jnp.inf, jnp.float32)
                ci[pl.ds(i * 16, 16)] = jnp.zeros((16,), jnp.int32)
                return c

            lax.fori_loop(0, M // 16, body3, 0)

            def body4(g, carry):
                off, sume = carry
                x = buf[g >> 3, pl.ds((g & 7) * 16, 16)]
                bkt, _ = bucket_of(x)
                m = bkt >= bnd
                plsc.store_compressed(cv.at[pl.ds(off, 16)], x, mask=m)
                gidx = g * 16 + lanes
                plsc.store_compressed(ci.at[pl.ds(off, 16)], gidx, mask=m)
                cnt = jnp.max(plsc.all_reduce_population_count(m))
                off = jnp.minimum(off + cnt, M - 16)
                sume = sume + jnp.exp(x - mx)
                return off, sume

            off, sume = lax.fori_loop(
                0, VP // 16, body4,
                (jnp.int32(0), jnp.zeros((16,), jnp.float32)), unroll=8)
            se = jnp.sum(sume)

            stv = jnp.where(lanes == 0, mx, jnp.where(lanes == 1, se, 0.0))
            st[pl.ds(0, 16)] = stv

            pltpu.sync_copy(cv, cv_hbm.at[pl.ds(row * M, M)])
            pltpu.sync_copy(ci, ci_hbm.at[pl.ds(row * M, M)])
            pltpu.sync_copy(st, st_hbm.at[pl.ds(row * 16, 16)])
            return 0

        lax.fori_loop(0, rows_per_w, row_body, 0)

    lp = jnp.pad(logits, ((0, 0), (0, VP - V)), constant_values=-jnp.inf)
    cv, ci, st = k(lp.reshape(B * VP // 128, 128))
    return cv.reshape(B, M), ci.reshape(B, M), st.reshape(B, 16)


BR = 8


def _tc_body(cv_ref, ci_ref, st_ref, tp_ref, tk_ref, mp_ref, sm_ref, g_ref,
             fp_ref, fi_ref, tok_ref):
    p0 = jnp.exp(cv_ref[...] - st_ref[:, 0:1]) / st_ref[:, 1:2]
    idx0 = ci_ref[...]

    lane_m = lax.broadcasted_iota(jnp.int32, (BR, M), 1)

    def stage(p, idx, k, j):
        is_lower = (lane_m & j) == 0
        desc = (lane_m & k) == 0
        pa = jnp.where(is_lower, pltpu.roll(p, M - j, 1), pltpu.roll(p, j, 1))
        ia = jnp.where(is_lower, pltpu.roll(idx, M - j, 1), pltpu.roll(idx, j, 1))
        gt = (p > pa) | ((p == pa) & (idx < ia))
        keep_self = (is_lower == desc) == gt
        return jnp.where(keep_self, p, pa), jnp.where(keep_self, idx, ia)

    def outer(kk, carry):
        k = jnp.int32(1) << kk

        def inner(t, carry):
            p, idx = carry
            j = jnp.int32(1) << (kk - 1 - t)
            return stage(p, idx, k, j)

        return lax.fori_loop(0, kk, inner, carry)

    p, idx = lax.fori_loop(1, 13, outer, (p0, idx0))

    ps = p[:, :K]
    pi = idx[:, :K]

    cs = ps
    s = 1
    while s < K:
        shifted = jnp.concatenate(
            [jnp.zeros((BR, s), jnp.float32), cs[:, :K - s]], axis=1)
        cs = cs + shifted
        s *= 2

    lane = lax.broadcasted_iota(jnp.int32, (BR, K), 1)
    tk = jnp.maximum(tk_ref[...], 1)
    tp = tp_ref[...]
    mp = mp_ref[...]
    apply_min_p = mp > 0.0
    mask_k = lane < tk
    mask_p = jnp.logical_not(cs - ps > tp)
    thr = jnp.where(apply_min_p, ps[:, 0:1] * mp, 0.0)
    fm = mask_k & mask_p
    fm = fm & jnp.logical_not(apply_min_p & (ps < thr))
    filtered = jnp.where(fm, ps, 0.0)
    denom = jnp.sum(filtered, axis=1, keepdims=True)
    denom_safe = jnp.where(denom == 0.0, 1.0, denom)
    normed = filtered / denom_safe
    normed = jnp.where((denom == 0.0) & (lane == 0), 1.0, normed)

    logp = jnp.where(normed > 0.0, jnp.log(jnp.maximum(normed, 1e-38)), -jnp.inf)
    scores = logp + g_ref[...]
    mval = jnp.max(scores, axis=1, keepdims=True)
    hit = jnp.where(scores == mval, lane, K)
    sampled = jnp.min(hit, axis=1, keepdims=True)
    onehot = lane == sampled
    tok = jnp.sum(jnp.where(onehot, pi, 0), axis=1, keepdims=True)

    std_probs = jnp.where(onehot, 1.0, 0.0)
    std_idx = jnp.where(onehot, tok, 0)
    sm = sm_ref[...] != 0
    fp_ref[...] = jnp.where(sm, normed, std_probs)
    fi_ref[...] = jnp.where(sm, pi, std_idx)
    tok_ref[...] = tok


def _tc_finish(cv, ci, st, tp, tk, mp, sm, g):
    def rows(cols):
        return pl.BlockSpec((BR, cols), lambda i: (i, 0))

    return pl.pallas_call(
        _tc_body,
        grid=(B // BR,),
        in_specs=[rows(M), rows(M), rows(16), rows(1), rows(1), rows(1),
                  rows(1), rows(K)],
        out_specs=[rows(K), rows(K), rows(1)],
        out_shape=[
            jax.ShapeDtypeStruct((B, K), jnp.float32),
            jax.ShapeDtypeStruct((B, K), jnp.int32),
            jax.ShapeDtypeStruct((B, 1), jnp.int32),
        ],
    )(cv, ci, st, tp, tk, mp, sm, g)


def kernel(logits, top_p, top_k, min_p, soft_mask):
    cv, ci, st = _sc_select(logits)
    g = jax.random.gumbel(jax.random.key(1234), (B, K), jnp.float32)
    fp, fi, tok = _tc_finish(
        cv, ci, st,
        top_p.reshape(B, 1), top_k.reshape(B, 1).astype(jnp.int32),
        min_p.reshape(B, 1), soft_mask.reshape(B, 1).astype(jnp.int32), g)
    return fp, fi, tok.reshape(B)


# speedup vs baseline: 1.2884x; 1.2884x over previous
import functools

import jax
import jax.numpy as jnp
from jax import lax
from jax.experimental import pallas as pl
from jax.experimental.pallas import tpu as pltpu
from jax.experimental.pallas import tpu_sc as plsc

B = 128
V = 100000
VP = 102400
K = 2048
M = 4096
NB = 4096
CH = 10240
NCH = VP // CH
GRP = CH // 16
CHR = CH // 128
HKEY = 0x7FFFFFFF


def _sc_select(logits):
    info = plsc.get_sparse_core_info()
    nc, ns = info.num_cores, info.num_subcores
    nw = nc * ns
    rows_per_w = B // nw
    mesh = plsc.VectorSubcoreMesh(core_axis_name="c", subcore_axis_name="s")

    @functools.partial(
        pl.kernel,
        mesh=mesh,
        compiler_params=pltpu.CompilerParams(needs_layout_passes=False),
        out_type=[
            jax.ShapeDtypeStruct((B * K,), jnp.float32),
            jax.ShapeDtypeStruct((B * K,), jnp.int32),
            jax.ShapeDtypeStruct((B * 16,), jnp.float32),
        ],
        scratch_types=[
            pltpu.VMEM((VP // 128, 128), jnp.float32),
            pltpu.VMEM((NB,), jnp.int32),
            pltpu.VMEM((M,), jnp.float32),
            pltpu.VMEM((M,), jnp.int32),
            pltpu.VMEM((16,), jnp.float32),
            pltpu.VMEM((M,), jnp.int32),
            pltpu.VMEM((M,), jnp.int32),
            pltpu.VMEM((M,), jnp.int32),
            pltpu.SemaphoreType.DMA((4,)),
        ],
    )
    def k(logits_hbm, cv_hbm, ci_hbm, st_hbm,
          buf, hist, cv, ci, st, kb0, kb1, ib, sem):
        wid = lax.axis_index("s") * nc + lax.axis_index("c")
        lanes = lax.iota(jnp.int32, 16)
        ones = jnp.ones((16,), jnp.int32)

        def zero_hist(i, c):
            hist[pl.ds(i * 16, 16)] = jnp.zeros((16,), jnp.int32)
            return c

        lax.fori_loop(0, NB // 16, zero_hist, 0)

        def bucket_of(x):
            bits = lax.bitcast_convert_type(x, jnp.int32)
            key = bits ^ ((bits >> 31) & HKEY)
            return (key >> 20) + 2048, key

        def start_copy(row2d, c):
            pltpu.make_async_copy(
                logits_hbm.at[pl.ds(row2d + c * CHR, CHR), :],
                buf.at[pl.ds(c * CHR, CHR), :], sem.at[c % 4]).start()

        def wait_copy(c):
            pltpu.make_async_copy(
                logits_hbm.at[pl.ds(0, CHR), :],
                buf.at[pl.ds(c * CHR, CHR), :], sem.at[c % 4]).wait()

        def row_body(r, _unused):
            row = wid * rows_per_w + r
            rowbase = row * (VP // 128)

            for cc in range(4):
                start_copy(rowbase, cc)

            def chunk1(c, vmax):
                @pl.when(c + 4 < NCH)
                def _():
                    start_copy(rowbase, c + 4)

                wait_copy(c)

                def body1(g, vmax):
                    gg = c * GRP + g
                    x = buf[gg >> 3, pl.ds((gg & 7) * 16, 16)]
                    bkt, _ = bucket_of(x)
                    plsc.addupdate_scatter(hist, [bkt], ones)
                    return jnp.maximum(vmax, x)

                return lax.fori_loop(0, GRP, body1, vmax, unroll=8)

            vmax = lax.fori_loop(
                0, NCH, chunk1, jnp.full((16,), -jnp.inf, jnp.float32))
            mx = jnp.max(vmax)

            def body2(t_rev, carry):
                acc, bnd, found = carry
                t = (NB // 16 - 1) - t_rev
                s = hist[pl.ds(t * 16, 16)]
                hist[pl.ds(t * 16, 16)] = jnp.zeros((16,), jnp.int32)
                cs = plsc.cumsum(s)
                total = jnp.max(cs)
                ssum = total - cs + s
                m = (acc + ssum) >= K
                cnt = jnp.max(plsc.all_reduce_population_count(m))
                newly = jnp.logical_and(jnp.logical_not(found), cnt > 0)
                bnd = jnp.where(newly, t * 16 + cnt - 1, bnd)
                found = jnp.logical_or(found, cnt > 0)
                return acc + total, bnd, found

            _, bnd, _ = lax.fori_loop(
                0, NB // 16, body2, (jnp.int32(0), jnp.int32(0), False))

            def body3(i, c):
                kb0[pl.ds(i * 16, 16)] = jnp.zeros((16,), jnp.int32)
                ci[pl.ds(i * 16, 16)] = jnp.zeros((16,), jnp.int32)
                return c

            lax.fori_loop(0, M // 16, body3, 0)

            def body4(g, carry):
                off, sume = carry
                x = buf[g >> 3, pl.ds((g & 7) * 16, 16)]
                bkt, key = bucket_of(x)
                ukey = key ^ jnp.int32(-2147483648)
                m = bkt >= bnd
                plsc.store_compressed(kb0.at[pl.ds(off, 16)], ukey, mask=m)
                gidx = g * 16 + lanes
                plsc.store_compressed(ci.at[pl.ds(off, 16)], gidx, mask=m)
                cnt = jnp.max(plsc.all_reduce_population_count(m))
                off = jnp.minimum(off + cnt, M - 16)
                sume = sume + jnp.exp(x - mx)
                return off, sume

            off, sume = lax.fori_loop(
                0, VP // 16, body4,
                (jnp.int32(0), jnp.zeros((16,), jnp.float32)), unroll=8)
            se = jnp.sum(sume)

            stv = jnp.where(lanes == 0, mx, jnp.where(lanes == 1, se, 0.0))
            st[pl.ds(0, 16)] = stv

            def digit_pass(p, srck, srci, dstk, dsti):
                sh = 5 * p
                hist[pl.ds(0, 16)] = jnp.zeros((16,), jnp.int32)
                hist[pl.ds(16, 16)] = jnp.zeros((16,), jnp.int32)

                def cnt_body(g, c):
                    k16 = srck[pl.ds(g * 16, 16)]
                    d = 31 - ((k16 >> sh) & 31)
                    plsc.addupdate_scatter(hist, [d], ones)
                    return c

                lax.fori_loop(0, M // 16, cnt_body, 0, unroll=8)

                h0 = hist[pl.ds(0, 16)]
                h1 = hist[pl.ds(16, 16)]
                cs0 = plsc.cumsum(h0)
                cs1 = plsc.cumsum(h1)
                t0 = jnp.max(cs0)
                hist[pl.ds(0, 16)] = cs0 - h0
                hist[pl.ds(16, 16)] = cs1 - h1 + t0

                def perm_body(g, c):
                    k16 = srck[pl.ds(g * 16, 16)]
                    i16 = srci[pl.ds(g * 16, 16)]
                    d = 31 - ((k16 >> sh) & 31)
                    rank = plsc.scan_count(d)[0] - 1
                    base = plsc.load_gather(hist, [d])
                    pos = base + rank
                    plsc.store_scatter(dstk, [pos], k16)
                    plsc.store_scatter(dsti, [pos], i16)
                    plsc.addupdate_scatter(hist, [d], ones)
                    return c

                lax.fori_loop(0, M // 16, perm_body, 0, unroll=4)

            for p in range(7):
                if p % 2 == 0:
                    digit_pass(p, kb0, ci, kb1, ib)
                else:
                    digit_pass(p, kb1, ib, kb0, ci)

            def out_body(g, c):
                u = kb1[pl.ds(g * 16, 16)]
                bits = jnp.where(u < 0, u ^ jnp.int32(-2147483648), ~u)
                cv[pl.ds(g * 16, 16)] = lax.bitcast_convert_type(
                    bits, jnp.float32)
                return c

            lax.fori_loop(0, K // 16, out_body, 0, unroll=8)

            pltpu.sync_copy(cv.at[pl.ds(0, K)], cv_hbm.at[pl.ds(row * K, K)])
            pltpu.sync_copy(ib.at[pl.ds(0, K)], ci_hbm.at[pl.ds(row * K, K)])
            pltpu.sync_copy(st, st_hbm.at[pl.ds(row * 16, 16)])
            return 0

        lax.fori_loop(0, rows_per_w, row_body, 0)

    lp = jnp.pad(logits, ((0, 0), (0, VP - V)), constant_values=-jnp.inf)
    cv, ci, st = k(lp.reshape(B * VP // 128, 128))
    return cv.reshape(B, K), ci.reshape(B, K), st.reshape(B, 16)


BR = 8


def _tc_body(cv_ref, ci_ref, st_ref, tp_ref, tk_ref, mp_ref, sm_ref, g_ref,
             fp_ref, fi_ref, tok_ref):
    ps = jnp.exp(cv_ref[...] - st_ref[:, 0:1]) / st_ref[:, 1:2]
    pi = ci_ref[...]

    cs = ps
    s = 1
    while s < K:
        shifted = jnp.concatenate(
            [jnp.zeros((BR, s), jnp.float32), cs[:, :K - s]], axis=1)
        cs = cs + shifted
        s *= 2

    lane = lax.broadcasted_iota(jnp.int32, (BR, K), 1)
    tk = jnp.maximum(tk_ref[...], 1)
    tp = tp_ref[...]
    mp = mp_ref[...]
    apply_min_p = mp > 0.0
    mask_k = lane < tk
    mask_p = jnp.logical_not(cs - ps > tp)
    thr = jnp.where(apply_min_p, ps[:, 0:1] * mp, 0.0)
    fm = mask_k & mask_p
    fm = fm & jnp.logical_not(apply_min_p & (ps < thr))
    filtered = jnp.where(fm, ps, 0.0)
    denom = jnp.sum(filtered, axis=1, keepdims=True)
    denom_safe = jnp.where(denom == 0.0, 1.0, denom)
    normed = filtered / denom_safe
    normed = jnp.where((denom == 0.0) & (lane == 0), 1.0, normed)

    logp = jnp.where(normed > 0.0, jnp.log(jnp.maximum(normed, 1e-38)), -jnp.inf)
    scores = logp + g_ref[...]
    mval = jnp.max(scores, axis=1, keepdims=True)
    hit = jnp.where(scores == mval, lane, K)
    sampled = jnp.min(hit, axis=1, keepdims=True)
    onehot = lane == sampled
    tok = jnp.sum(jnp.where(onehot, pi, 0), axis=1, keepdims=True)

    std_probs = jnp.where(onehot, 1.0, 0.0)
    std_idx = jnp.where(onehot, tok, 0)
    sm = sm_ref[...] != 0
    fp_ref[...] = jnp.where(sm, normed, std_probs)
    fi_ref[...] = jnp.where(sm, pi, std_idx)
    tok_ref[...] = tok


def _tc_finish(cv, ci, st, tp, tk, mp, sm, g):
    def rows(cols):
        return pl.BlockSpec((BR, cols), lambda i: (i, 0))

    return pl.pallas_call(
        _tc_body,
        grid=(B // BR,),
        in_specs=[rows(K), rows(K), rows(16), rows(1), rows(1), rows(1),
                  rows(1), rows(K)],
        out_specs=[rows(K), rows(K), rows(1)],
        out_shape=[
            jax.ShapeDtypeStruct((B, K), jnp.float32),
            jax.ShapeDtypeStruct((B, K), jnp.int32),
            jax.ShapeDtypeStruct((B, 1), jnp.int32),
        ],
    )(cv, ci, st, tp, tk, mp, sm, g)


def kernel(logits, top_p, top_k, min_p, soft_mask):
    cv, ci, st = _sc_select(logits)
    g = jax.random.gumbel(jax.random.key(1234), (B, K), jnp.float32)
    fp, fi, tok = _tc_finish(
        cv, ci, st,
        top_p.reshape(B, 1), top_k.reshape(B, 1).astype(jnp.int32),
        min_p.reshape(B, 1), soft_mask.reshape(B, 1).astype(jnp.int32), g)
    return fp, fi, tok.reshape(B)
